# Initial kernel scaffold; baseline (speedup 1.0000x reference)
import jax
import jax.numpy as jnp
from jax.experimental import pallas as pl


def kernel(drug_x, drug_seq_x, target_x, target_seq_x, params, drug_edge_index, drug_batch, target_edge_index, target_batch):
    raise NotImplementedError("write your pallas kernel here")


# speedup vs baseline: 1.0104x; 1.0104x over previous
import jax
import jax.numpy as jnp
from jax.experimental import pallas as pl
from jax.experimental.pallas import tpu as pltpu

N_D = 50000
N_T = 50000
G = 512


def _gcn_conv_jax(x, src, dst, W, b, n):
    loop = jnp.arange(n, dtype=src.dtype)
    s = jnp.concatenate([src, loop])
    d = jnp.concatenate([dst, loop])
    h = x @ W
    deg = jnp.zeros((n,), x.dtype).at[d].add(1.0)
    dinv = jnp.where(deg > 0, 1.0 / jnp.sqrt(deg), 0.0)
    norm = dinv[s] * dinv[d]
    out = jnp.zeros_like(h).at[d].add(norm[:, None] * h[s])
    return out + b


def _seg_mean_jax(x, seg, n_seg):
    s = jax.ops.segment_sum(x, seg, num_segments=n_seg)
    c = jax.ops.segment_sum(jnp.ones((x.shape[0],), x.dtype), seg,
                            num_segments=n_seg)
    return s / jnp.clip(c, 1.0)[:, None]


def _res_tower(x, ws, bs):
    out = jax.nn.relu(jnp.dot(x, ws[0], preferred_element_type=jnp.float32) + bs[0])
    last = out
    for i in range(8):
        if i != 0 and i % 2 == 0:
            out = out + last
        out = jax.nn.relu(jnp.dot(out, ws[1 + i], preferred_element_type=jnp.float32) + bs[1 + i])
        if i != 0 and i % 2 == 0:
            last = out
    return jnp.dot(out, ws[9], preferred_element_type=jnp.float32) + bs[9]


def _heads_kernel(dg_ref, ds_ref, tg_ref, ts_ref, *refs):
    refs = list(refs)
    dsw = [refs[i][...] for i in range(10)]
    dsb = [refs[10 + i][...] for i in range(10)]
    tsw = [refs[20 + i][...] for i in range(10)]
    tsb = [refs[30 + i][...] for i in range(10)]
    dow = [refs[40][...], refs[41][...]]
    dob = [refs[42][...], refs[43][...]]
    tow = [refs[44][...], refs[45][...]]
    tob = [refs[46][...], refs[47][...]]
    d_out_ref, t_out_ref = refs[48], refs[49]

    ds_emb = _res_tower(ds_ref[...], dsw, dsb)
    ts_emb = _res_tower(ts_ref[...], tsw, tsb)

    dcat = jnp.concatenate([dg_ref[...], ds_emb], axis=-1)
    h = jax.nn.relu(jnp.dot(dcat, dow[0], preferred_element_type=jnp.float32) + dob[0])
    d_out_ref[...] = jnp.dot(h, dow[1], preferred_element_type=jnp.float32) + dob[1]

    tcat = jnp.concatenate([tg_ref[...], ts_emb], axis=-1)
    h = jax.nn.relu(jnp.dot(tcat, tow[0], preferred_element_type=jnp.float32) + tob[0])
    t_out_ref[...] = jnp.dot(h, tow[1], preferred_element_type=jnp.float32) + tob[1]


def _run_heads(drug_graph_emb, drug_seq_x, target_graph_emb, target_seq_x, params):
    ds_w = [params['ds_W1']] + [params['ds_Wh%d' % i] for i in range(8)] + [params['ds_W2']]
    ds_b = [params['ds_b1']] + [params['ds_bh%d' % i] for i in range(8)] + [params['ds_b2']]
    ts_w = [params['ts_W1']] + [params['ts_Wh%d' % i] for i in range(8)] + [params['ts_W2']]
    ts_b = [params['ts_b1']] + [params['ts_bh%d' % i] for i in range(8)] + [params['ts_b2']]
    args = ([drug_graph_emb, drug_seq_x, target_graph_emb, target_seq_x]
            + ds_w + ds_b + ts_w + ts_b
            + [params['do_W0'], params['do_W1'], params['do_b0'], params['do_b1']]
            + [params['to_W0'], params['to_W1'], params['to_b0'], params['to_b1']])
    out_shapes = (jax.ShapeDtypeStruct((G, 128), jnp.float32),
                  jax.ShapeDtypeStruct((G, 128), jnp.float32))
    return pl.pallas_call(
        _heads_kernel,
        out_shape=out_shapes,
    )(*args)


def kernel(drug_x, drug_seq_x, target_x, target_seq_x, params,
           drug_edge_index, drug_batch, target_edge_index, target_batch):
    h = drug_x
    for i in range(3):
        h = jax.nn.relu(_gcn_conv_jax(h, drug_edge_index[0], drug_edge_index[1],
                                      params['dg_W%d' % i], params['dg_b%d' % i], N_D))
    drug_graph_emb = _seg_mean_jax(h, drug_batch, G)
    h = target_x
    for i in range(3):
        h = jax.nn.relu(_gcn_conv_jax(h, target_edge_index[0], target_edge_index[1],
                                      params['tg_W%d' % i], params['tg_b%d' % i], N_T))
    target_graph_emb = _seg_mean_jax(h, target_batch, G)
    drug_out, target_out = _run_heads(drug_graph_emb, drug_seq_x,
                                      target_graph_emb, target_seq_x, params)
    return drug_out, target_out


# speedup vs baseline: 4.9505x; 4.8995x over previous
import functools

import jax
import jax.numpy as jnp
from jax import lax
from jax.experimental import pallas as pl
from jax.experimental.pallas import tpu as pltpu
from jax.experimental.pallas import tpu_sc as plsc

N = 50000
NP = 50176
E = 800000
G = 512
FC = 32
NC, NS = 2, 16
NW = NC * NS
EPT = E // NW
CHUNK = 125
NCHUNK = EPT // CHUNK
RPT = NP // NS
ZROWS = 196
KB = 25


def _agg_body(nb, src_hbm, dst_hbm, *rest):
    g_refs = rest[:nb]
    out_ref = rest[nb]
    sidx, didx, rows, zbuf, acc, sem = rest[nb + 1:]
    c = lax.axis_index("c")
    s = lax.axis_index("s")
    w = c * NS + s

    zeros16 = jnp.zeros((16,), jnp.float32)

    def _zb(i, carry):
        zbuf[i, pl.ds(0, 16)] = zeros16
        zbuf[i, pl.ds(16, 16)] = zeros16
        return carry

    lax.fori_loop(0, ZROWS, _zb, 0)

    for j in range(nb):
        for z in range(RPT // ZROWS):
            pltpu.sync_copy(zbuf, acc.at[pl.ds(s * RPT + z * ZROWS, ZROWS)])
        plsc.subcore_barrier()

        g_ref = g_refs[j]

        def _kblock(kb, carry):
            pltpu.sync_copy(src_hbm.at[w, pl.ds(kb * KB, KB)], sidx)
            pltpu.sync_copy(dst_hbm.at[w, pl.ds(kb * KB, KB)], didx)

            def _chunk(i, c2):
                pltpu.async_copy(g_ref.at[sidx.at[i]], rows, sem).wait()
                pltpu.sync_copy(rows, acc.at[didx.at[i]], add=True)
                return c2

            lax.fori_loop(0, KB, _chunk, 0)
            return carry

        lax.fori_loop(0, NCHUNK // KB, _kblock, 0)
        plsc.subcore_barrier()

        pltpu.sync_copy(acc.at[pl.ds(s * RPT, RPT)],
                        out_ref.at[c, j, pl.ds(s * RPT, RPT)])
        plsc.subcore_barrier()


@functools.lru_cache(maxsize=None)
def _make_agg(nb):
    mesh = plsc.VectorSubcoreMesh(core_axis_name="c", subcore_axis_name="s")
    return pl.kernel(
        functools.partial(_agg_body, nb),
        out_type=jax.ShapeDtypeStruct((NC, nb, NP, FC), jnp.float32),
        mesh=mesh,
        scratch_types=[
            pltpu.VMEM((KB, CHUNK), jnp.int32),
            pltpu.VMEM((KB, CHUNK), jnp.int32),
            pltpu.VMEM((CHUNK, FC), jnp.float32),
            pltpu.VMEM((ZROWS, FC), jnp.float32),
            pltpu.VMEM_SHARED((NP, FC), jnp.float32),
            pltpu.SemaphoreType.DMA,
        ],
        compiler_params=pltpu.CompilerParams(use_tc_tiling_on_sc=False),
        name="edge_agg_nb%d" % nb,
    )


def _edge_aggregate(g, src2, dst2, f_out):
    nb = -(-f_out // FC)
    f_pad = nb * FC
    g = jnp.pad(g, ((0, NP - N), (0, f_pad - f_out)))
    gblocks = [g[:, j * FC:(j + 1) * FC] for j in range(nb)]
    parts = _make_agg(nb)(src2, dst2, *gblocks)
    agg = parts[0] + parts[1]
    agg = agg.transpose(1, 0, 2).reshape(NP, f_pad)
    return agg[:N, :f_out]


def _res_tower(x, ws, bs):
    out = jax.nn.relu(jnp.dot(x, ws[0], preferred_element_type=jnp.float32) + bs[0])
    last = out
    for i in range(8):
        if i != 0 and i % 2 == 0:
            out = out + last
        out = jax.nn.relu(jnp.dot(out, ws[1 + i], preferred_element_type=jnp.float32) + bs[1 + i])
        if i != 0 and i % 2 == 0:
            last = out
    return jnp.dot(out, ws[9], preferred_element_type=jnp.float32) + bs[9]


def _heads_kernel(dg_ref, ds_ref, tg_ref, ts_ref, *refs):
    refs = list(refs)
    dsw = [refs[i][...] for i in range(10)]
    dsb = [refs[10 + i][...] for i in range(10)]
    tsw = [refs[20 + i][...] for i in range(10)]
    tsb = [refs[30 + i][...] for i in range(10)]
    dow = [refs[40][...], refs[41][...]]
    dob = [refs[42][...], refs[43][...]]
    tow = [refs[44][...], refs[45][...]]
    tob = [refs[46][...], refs[47][...]]
    d_out_ref, t_out_ref = refs[48], refs[49]

    ds_emb = _res_tower(ds_ref[...], dsw, dsb)
    ts_emb = _res_tower(ts_ref[...], tsw, tsb)

    dcat = jnp.concatenate([dg_ref[...], ds_emb], axis=-1)
    h = jax.nn.relu(jnp.dot(dcat, dow[0], preferred_element_type=jnp.float32) + dob[0])
    d_out_ref[...] = jnp.dot(h, dow[1], preferred_element_type=jnp.float32) + dob[1]

    tcat = jnp.concatenate([tg_ref[...], ts_emb], axis=-1)
    h = jax.nn.relu(jnp.dot(tcat, tow[0], preferred_element_type=jnp.float32) + tob[0])
    t_out_ref[...] = jnp.dot(h, tow[1], preferred_element_type=jnp.float32) + tob[1]


def _run_heads(drug_graph_emb, drug_seq_x, target_graph_emb, target_seq_x, params):
    ds_w = [params['ds_W1']] + [params['ds_Wh%d' % i] for i in range(8)] + [params['ds_W2']]
    ds_b = [params['ds_b1']] + [params['ds_bh%d' % i] for i in range(8)] + [params['ds_b2']]
    ts_w = [params['ts_W1']] + [params['ts_Wh%d' % i] for i in range(8)] + [params['ts_W2']]
    ts_b = [params['ts_b1']] + [params['ts_bh%d' % i] for i in range(8)] + [params['ts_b2']]
    args = ([drug_graph_emb, drug_seq_x, target_graph_emb, target_seq_x]
            + ds_w + ds_b + ts_w + ts_b
            + [params['do_W0'], params['do_W1'], params['do_b0'], params['do_b1']]
            + [params['to_W0'], params['to_W1'], params['to_b0'], params['to_b1']])
    out_shapes = (jax.ShapeDtypeStruct((G, 128), jnp.float32),
                  jax.ShapeDtypeStruct((G, 128), jnp.float32))
    return pl.pallas_call(_heads_kernel, out_shape=out_shapes)(*args)


def _seg_mean_jax(x, seg, n_seg):
    s = jax.ops.segment_sum(x, seg, num_segments=n_seg)
    c = jax.ops.segment_sum(jnp.ones((x.shape[0],), x.dtype), seg,
                            num_segments=n_seg)
    return s / jnp.clip(c, 1.0)[:, None]


def _gcn_branch(x, edge_index, params, prefix, dims):
    src = edge_index[0].astype(jnp.int32)
    dst = edge_index[1].astype(jnp.int32)
    deg = jnp.zeros((N,), jnp.float32).at[dst].add(1.0) + 1.0
    dinv = lax.rsqrt(deg)
    src2 = src.reshape(NW, NCHUNK, CHUNK)
    dst2 = dst.reshape(NW, NCHUNK, CHUNK)
    h_in = x
    for i in range(3):
        w = params['%s_W%d' % (prefix, i)]
        b = params['%s_b%d' % (prefix, i)]
        h = jnp.dot(h_in, w, preferred_element_type=jnp.float32)
        g = dinv[:, None] * h
        agg = _edge_aggregate(g, src2, dst2, dims[i + 1])
        h_in = jax.nn.relu(dinv[:, None] * agg + dinv[:, None] ** 2 * h + b)
    return h_in


def kernel(drug_x, drug_seq_x, target_x, target_seq_x, params,
           drug_edge_index, drug_batch, target_edge_index, target_batch):
    hd = _gcn_branch(drug_x, drug_edge_index, params, 'dg', [78, 78, 156, 312])
    drug_graph_emb = _seg_mean_jax(hd, drug_batch, G)
    ht = _gcn_branch(target_x, target_edge_index, params, 'tg', [54, 54, 108, 216])
    target_graph_emb = _seg_mean_jax(ht, target_batch, G)
    drug_out, target_out = _run_heads(drug_graph_emb, drug_seq_x,
                                      target_graph_emb, target_seq_x, params)
    return drug_out, target_out


# speedup vs baseline: 7.0805x; 1.4302x over previous
import functools

import jax
import jax.numpy as jnp
from jax import lax
from jax.experimental import pallas as pl
from jax.experimental.pallas import tpu as pltpu
from jax.experimental.pallas import tpu_sc as plsc

N = 50000
NP = 50176
E = 800000
G = 512
FC = 32
NC, NS = 2, 16
NW = NC * NS
EPT = E // NW
CHUNK = 125
NCHUNK = EPT // CHUNK
RPT = NP // NS
ZROWS = 196
KB = 20
NBUF = 4


def _agg_body(nb, src_hbm, dst_hbm, *rest):
    g_refs = rest[:nb]
    out_ref = rest[nb]
    (sidx, didx, rows0, rows1, rows2, rows3, zbuf, acc,
     gsem, ssem, csem) = rest[nb + 1:]
    rowbufs = (rows0, rows1, rows2, rows3)
    c = lax.axis_index("c")
    s = lax.axis_index("s")
    w = c * NS + s

    zeros16 = jnp.zeros((16,), jnp.float32)

    def _zb(i, carry):
        zbuf[i, pl.ds(0, 16)] = zeros16
        zbuf[i, pl.ds(16, 16)] = zeros16
        return carry

    lax.fori_loop(0, ZROWS, _zb, 0)

    for j in range(nb):
        for z in range(RPT // ZROWS):
            pltpu.sync_copy(zbuf, acc.at[pl.ds(s * RPT + z * ZROWS, ZROWS)])
        plsc.subcore_barrier()

        g_ref = g_refs[j]

        def _kblock(kb, carry):
            pltpu.async_copy(src_hbm.at[w, pl.ds(kb * KB, KB)], sidx, csem)
            pltpu.async_copy(dst_hbm.at[w, pl.ds(kb * KB, KB)], didx,
                             csem).wait()
            pltpu.make_async_copy(src_hbm.at[w, pl.ds(kb * KB, KB)], sidx,
                                  csem).wait()

            def _batch(p, c2):
                i0 = p * NBUF
                gathers = [
                    pltpu.async_copy(g_ref.at[sidx.at[i0 + b]], rowbufs[b],
                                     gsem)
                    for b in range(NBUF)
                ]
                scatters = []
                for b in range(NBUF):
                    gathers[b].wait()
                    scatters.append(
                        pltpu.async_copy(rowbufs[b],
                                         acc.at[didx.at[i0 + b]], ssem,
                                         add=True))
                for d in scatters:
                    d.wait()
                return c2

            lax.fori_loop(0, KB // NBUF, _batch, 0)
            return carry

        lax.fori_loop(0, NCHUNK // KB, _kblock, 0)
        plsc.subcore_barrier()

        pltpu.sync_copy(acc.at[pl.ds(s * RPT, RPT)],
                        out_ref.at[c, j, pl.ds(s * RPT, RPT)])
        plsc.subcore_barrier()


@functools.lru_cache(maxsize=None)
def _make_agg(nb):
    mesh = plsc.VectorSubcoreMesh(core_axis_name="c", subcore_axis_name="s")
    return pl.kernel(
        functools.partial(_agg_body, nb),
        out_type=jax.ShapeDtypeStruct((NC, nb, NP, FC), jnp.float32),
        mesh=mesh,
        scratch_types=[
            pltpu.VMEM((KB, CHUNK), jnp.int32),
            pltpu.VMEM((KB, CHUNK), jnp.int32),
            pltpu.VMEM((CHUNK, FC), jnp.float32),
            pltpu.VMEM((CHUNK, FC), jnp.float32),
            pltpu.VMEM((CHUNK, FC), jnp.float32),
            pltpu.VMEM((CHUNK, FC), jnp.float32),
            pltpu.VMEM((ZROWS, FC), jnp.float32),
            pltpu.VMEM_SHARED((NP, FC), jnp.float32),
            pltpu.SemaphoreType.DMA,
            pltpu.SemaphoreType.DMA,
            pltpu.SemaphoreType.DMA,
        ],
        compiler_params=pltpu.CompilerParams(use_tc_tiling_on_sc=False),
        name="edge_agg_nb%d" % nb,
    )


def _edge_aggregate(g, src2, dst2, f_out):
    nb = -(-f_out // FC)
    f_pad = nb * FC
    g = jnp.pad(g, ((0, NP - N), (0, f_pad - f_out)))
    gblocks = [g[:, j * FC:(j + 1) * FC] for j in range(nb)]
    parts = _make_agg(nb)(src2, dst2, *gblocks)
    agg = parts[0] + parts[1]
    agg = agg.transpose(1, 0, 2).reshape(NP, f_pad)
    return agg[:N, :f_out]


def _res_tower(x, ws, bs):
    out = jax.nn.relu(jnp.dot(x, ws[0], preferred_element_type=jnp.float32) + bs[0])
    last = out
    for i in range(8):
        if i != 0 and i % 2 == 0:
            out = out + last
        out = jax.nn.relu(jnp.dot(out, ws[1 + i], preferred_element_type=jnp.float32) + bs[1 + i])
        if i != 0 and i % 2 == 0:
            last = out
    return jnp.dot(out, ws[9], preferred_element_type=jnp.float32) + bs[9]


def _heads_kernel(dg_ref, ds_ref, tg_ref, ts_ref, *refs):
    refs = list(refs)
    dsw = [refs[i][...] for i in range(10)]
    dsb = [refs[10 + i][...] for i in range(10)]
    tsw = [refs[20 + i][...] for i in range(10)]
    tsb = [refs[30 + i][...] for i in range(10)]
    dow = [refs[40][...], refs[41][...]]
    dob = [refs[42][...], refs[43][...]]
    tow = [refs[44][...], refs[45][...]]
    tob = [refs[46][...], refs[47][...]]
    d_out_ref, t_out_ref = refs[48], refs[49]

    ds_emb = _res_tower(ds_ref[...], dsw, dsb)
    ts_emb = _res_tower(ts_ref[...], tsw, tsb)

    dcat = jnp.concatenate([dg_ref[...], ds_emb], axis=-1)
    h = jax.nn.relu(jnp.dot(dcat, dow[0], preferred_element_type=jnp.float32) + dob[0])
    d_out_ref[...] = jnp.dot(h, dow[1], preferred_element_type=jnp.float32) + dob[1]

    tcat = jnp.concatenate([tg_ref[...], ts_emb], axis=-1)
    h = jax.nn.relu(jnp.dot(tcat, tow[0], preferred_element_type=jnp.float32) + tob[0])
    t_out_ref[...] = jnp.dot(h, tow[1], preferred_element_type=jnp.float32) + tob[1]


def _run_heads(drug_graph_emb, drug_seq_x, target_graph_emb, target_seq_x, params):
    ds_w = [params['ds_W1']] + [params['ds_Wh%d' % i] for i in range(8)] + [params['ds_W2']]
    ds_b = [params['ds_b1']] + [params['ds_bh%d' % i] for i in range(8)] + [params['ds_b2']]
    ts_w = [params['ts_W1']] + [params['ts_Wh%d' % i] for i in range(8)] + [params['ts_W2']]
    ts_b = [params['ts_b1']] + [params['ts_bh%d' % i] for i in range(8)] + [params['ts_b2']]
    args = ([drug_graph_emb, drug_seq_x, target_graph_emb, target_seq_x]
            + ds_w + ds_b + ts_w + ts_b
            + [params['do_W0'], params['do_W1'], params['do_b0'], params['do_b1']]
            + [params['to_W0'], params['to_W1'], params['to_b0'], params['to_b1']])
    out_shapes = (jax.ShapeDtypeStruct((G, 128), jnp.float32),
                  jax.ShapeDtypeStruct((G, 128), jnp.float32))
    return pl.pallas_call(_heads_kernel, out_shape=out_shapes)(*args)


def _seg_mean_jax(x, seg, n_seg):
    s = jax.ops.segment_sum(x, seg, num_segments=n_seg)
    c = jax.ops.segment_sum(jnp.ones((x.shape[0],), x.dtype), seg,
                            num_segments=n_seg)
    return s / jnp.clip(c, 1.0)[:, None]


def _gcn_branch(x, edge_index, params, prefix, dims):
    src = edge_index[0].astype(jnp.int32)
    dst = edge_index[1].astype(jnp.int32)
    deg = jnp.zeros((N,), jnp.float32).at[dst].add(1.0) + 1.0
    dinv = lax.rsqrt(deg)
    src2 = src.reshape(NW, NCHUNK, CHUNK)
    dst2 = dst.reshape(NW, NCHUNK, CHUNK)
    h_in = x
    for i in range(3):
        w = params['%s_W%d' % (prefix, i)]
        b = params['%s_b%d' % (prefix, i)]
        h = jnp.dot(h_in, w, preferred_element_type=jnp.float32)
        g = dinv[:, None] * h
        agg = _edge_aggregate(g, src2, dst2, dims[i + 1])
        h_in = jax.nn.relu(dinv[:, None] * agg + dinv[:, None] ** 2 * h + b)
    return h_in


def kernel(drug_x, drug_seq_x, target_x, target_seq_x, params,
           drug_edge_index, drug_batch, target_edge_index, target_batch):
    hd = _gcn_branch(drug_x, drug_edge_index, params, 'dg', [78, 78, 156, 312])
    drug_graph_emb = _seg_mean_jax(hd, drug_batch, G)
    ht = _gcn_branch(target_x, target_edge_index, params, 'tg', [54, 54, 108, 216])
    target_graph_emb = _seg_mean_jax(ht, target_batch, G)
    drug_out, target_out = _run_heads(drug_graph_emb, drug_seq_x,
                                      target_graph_emb, target_seq_x, params)
    return drug_out, target_out


# speedup vs baseline: 7.2651x; 1.0261x over previous
import functools

import jax
import jax.numpy as jnp
from jax import lax
from jax.experimental import pallas as pl
from jax.experimental.pallas import tpu as pltpu
from jax.experimental.pallas import tpu_sc as plsc

N = 50000
NP = 50176
E = 800000
G = 512
FC = 32
NC, NS = 2, 16
NW = NC * NS
EPT = E // NW
CHUNK = 125
NCHUNK = EPT // CHUNK
RPT = NP // NS
ZROWS = 196
KB = 20
NBUF = 4


def _agg_body(nb, src_hbm, dst_hbm, *rest):
    g_refs = rest[:nb]
    out_ref = rest[nb]
    (sidx, didx, rows0, rows1, rows2, rows3, zbuf, acc,
     gsem, ssem, csem) = rest[nb + 1:]
    rowbufs = (rows0, rows1, rows2, rows3)
    gsems = tuple(gsem.at[b] for b in range(NBUF))
    ssems = tuple(ssem.at[b] for b in range(NBUF))
    c = lax.axis_index("c")
    s = lax.axis_index("s")
    w = c * NS + s

    zeros16 = jnp.zeros((16,), jnp.float32)

    def _zb(i, carry):
        zbuf[i, pl.ds(0, 16)] = zeros16
        zbuf[i, pl.ds(16, 16)] = zeros16
        return carry

    lax.fori_loop(0, ZROWS, _zb, 0)

    for j in range(nb):
        for z in range(RPT // ZROWS):
            pltpu.sync_copy(zbuf, acc.at[pl.ds(s * RPT + z * ZROWS, ZROWS)])
        plsc.subcore_barrier()

        g_ref = g_refs[j]

        def _kblock(kb, carry):
            pltpu.async_copy(src_hbm.at[w, pl.ds(kb * KB, KB)], sidx, csem)
            pltpu.async_copy(dst_hbm.at[w, pl.ds(kb * KB, KB)], didx,
                             csem).wait()
            pltpu.make_async_copy(src_hbm.at[w, pl.ds(kb * KB, KB)], sidx,
                                  csem).wait()

            def _batch(p, c2):
                i0 = p * NBUF
                gathers = []
                for b in range(NBUF):
                    @pl.when(p > 0)
                    def _drain(b=b, i0=i0):
                        pltpu.make_async_copy(
                            rowbufs[b], acc.at[didx.at[i0 + b]],
                            ssems[b]).wait()
                    gathers.append(
                        pltpu.async_copy(g_ref.at[sidx.at[i0 + b]],
                                         rowbufs[b], gsems[b]))
                for b in range(NBUF):
                    gathers[b].wait()
                    pltpu.async_copy(rowbufs[b], acc.at[didx.at[i0 + b]],
                                     ssems[b], add=True)
                return c2

            lax.fori_loop(0, KB // NBUF, _batch, 0)
            last = KB - NBUF
            for b in range(NBUF):
                pltpu.make_async_copy(rowbufs[b], acc.at[didx.at[last + b]],
                                      ssems[b]).wait()
            return carry

        lax.fori_loop(0, NCHUNK // KB, _kblock, 0)
        plsc.subcore_barrier()

        pltpu.sync_copy(acc.at[pl.ds(s * RPT, RPT)],
                        out_ref.at[c, j, pl.ds(s * RPT, RPT)])
        plsc.subcore_barrier()


@functools.lru_cache(maxsize=None)
def _make_agg(nb):
    mesh = plsc.VectorSubcoreMesh(core_axis_name="c", subcore_axis_name="s")
    return pl.kernel(
        functools.partial(_agg_body, nb),
        out_type=jax.ShapeDtypeStruct((NC, nb, NP, FC), jnp.float32),
        mesh=mesh,
        scratch_types=[
            pltpu.VMEM((KB, CHUNK), jnp.int32),
            pltpu.VMEM((KB, CHUNK), jnp.int32),
            pltpu.VMEM((CHUNK, FC), jnp.float32),
            pltpu.VMEM((CHUNK, FC), jnp.float32),
            pltpu.VMEM((CHUNK, FC), jnp.float32),
            pltpu.VMEM((CHUNK, FC), jnp.float32),
            pltpu.VMEM((ZROWS, FC), jnp.float32),
            pltpu.VMEM_SHARED((NP, FC), jnp.float32),
            pltpu.SemaphoreType.DMA((NBUF,)),
            pltpu.SemaphoreType.DMA((NBUF,)),
            pltpu.SemaphoreType.DMA,
        ],
        compiler_params=pltpu.CompilerParams(use_tc_tiling_on_sc=False),
        name="edge_agg_nb%d" % nb,
    )


def _edge_aggregate(g, src2, dst2, f_out):
    nb = -(-f_out // FC)
    f_pad = nb * FC
    g = jnp.pad(g, ((0, NP - N), (0, f_pad - f_out)))
    gblocks = [g[:, j * FC:(j + 1) * FC] for j in range(nb)]
    parts = _make_agg(nb)(src2, dst2, *gblocks)
    agg = parts[0] + parts[1]
    agg = agg.transpose(1, 0, 2).reshape(NP, f_pad)
    return agg[:N, :f_out]


def _res_tower(x, ws, bs):
    out = jax.nn.relu(jnp.dot(x, ws[0], preferred_element_type=jnp.float32) + bs[0])
    last = out
    for i in range(8):
        if i != 0 and i % 2 == 0:
            out = out + last
        out = jax.nn.relu(jnp.dot(out, ws[1 + i], preferred_element_type=jnp.float32) + bs[1 + i])
        if i != 0 and i % 2 == 0:
            last = out
    return jnp.dot(out, ws[9], preferred_element_type=jnp.float32) + bs[9]


def _heads_kernel(dg_ref, ds_ref, tg_ref, ts_ref, *refs):
    refs = list(refs)
    dsw = [refs[i][...] for i in range(10)]
    dsb = [refs[10 + i][...] for i in range(10)]
    tsw = [refs[20 + i][...] for i in range(10)]
    tsb = [refs[30 + i][...] for i in range(10)]
    dow = [refs[40][...], refs[41][...]]
    dob = [refs[42][...], refs[43][...]]
    tow = [refs[44][...], refs[45][...]]
    tob = [refs[46][...], refs[47][...]]
    d_out_ref, t_out_ref = refs[48], refs[49]

    ds_emb = _res_tower(ds_ref[...], dsw, dsb)
    ts_emb = _res_tower(ts_ref[...], tsw, tsb)

    dcat = jnp.concatenate([dg_ref[...], ds_emb], axis=-1)
    h = jax.nn.relu(jnp.dot(dcat, dow[0], preferred_element_type=jnp.float32) + dob[0])
    d_out_ref[...] = jnp.dot(h, dow[1], preferred_element_type=jnp.float32) + dob[1]

    tcat = jnp.concatenate([tg_ref[...], ts_emb], axis=-1)
    h = jax.nn.relu(jnp.dot(tcat, tow[0], preferred_element_type=jnp.float32) + tob[0])
    t_out_ref[...] = jnp.dot(h, tow[1], preferred_element_type=jnp.float32) + tob[1]


def _run_heads(drug_graph_emb, drug_seq_x, target_graph_emb, target_seq_x, params):
    ds_w = [params['ds_W1']] + [params['ds_Wh%d' % i] for i in range(8)] + [params['ds_W2']]
    ds_b = [params['ds_b1']] + [params['ds_bh%d' % i] for i in range(8)] + [params['ds_b2']]
    ts_w = [params['ts_W1']] + [params['ts_Wh%d' % i] for i in range(8)] + [params['ts_W2']]
    ts_b = [params['ts_b1']] + [params['ts_bh%d' % i] for i in range(8)] + [params['ts_b2']]
    args = ([drug_graph_emb, drug_seq_x, target_graph_emb, target_seq_x]
            + ds_w + ds_b + ts_w + ts_b
            + [params['do_W0'], params['do_W1'], params['do_b0'], params['do_b1']]
            + [params['to_W0'], params['to_W1'], params['to_b0'], params['to_b1']])
    out_shapes = (jax.ShapeDtypeStruct((G, 128), jnp.float32),
                  jax.ShapeDtypeStruct((G, 128), jnp.float32))
    return pl.pallas_call(_heads_kernel, out_shape=out_shapes)(*args)


def _seg_mean_jax(x, seg, n_seg):
    s = jax.ops.segment_sum(x, seg, num_segments=n_seg)
    c = jax.ops.segment_sum(jnp.ones((x.shape[0],), x.dtype), seg,
                            num_segments=n_seg)
    return s / jnp.clip(c, 1.0)[:, None]


def _gcn_branch(x, edge_index, params, prefix, dims):
    src = edge_index[0].astype(jnp.int32)
    dst = edge_index[1].astype(jnp.int32)
    deg = jnp.zeros((N,), jnp.float32).at[dst].add(1.0) + 1.0
    dinv = lax.rsqrt(deg)
    src2 = src.reshape(NW, NCHUNK, CHUNK)
    dst2 = dst.reshape(NW, NCHUNK, CHUNK)
    h_in = x
    for i in range(3):
        w = params['%s_W%d' % (prefix, i)]
        b = params['%s_b%d' % (prefix, i)]
        h = jnp.dot(h_in, w, preferred_element_type=jnp.float32)
        g = dinv[:, None] * h
        agg = _edge_aggregate(g, src2, dst2, dims[i + 1])
        h_in = jax.nn.relu(dinv[:, None] * agg + dinv[:, None] ** 2 * h + b)
    return h_in


def kernel(drug_x, drug_seq_x, target_x, target_seq_x, params,
           drug_edge_index, drug_batch, target_edge_index, target_batch):
    hd = _gcn_branch(drug_x, drug_edge_index, params, 'dg', [78, 78, 156, 312])
    drug_graph_emb = _seg_mean_jax(hd, drug_batch, G)
    ht = _gcn_branch(target_x, target_edge_index, params, 'tg', [54, 54, 108, 216])
    target_graph_emb = _seg_mean_jax(ht, target_batch, G)
    drug_out, target_out = _run_heads(drug_graph_emb, drug_seq_x,
                                      target_graph_emb, target_seq_x, params)
    return drug_out, target_out


# speedup vs baseline: 9.4948x; 1.3069x over previous
import functools

import jax
import jax.numpy as jnp
from jax import lax
from jax.experimental import pallas as pl
from jax.experimental.pallas import tpu as pltpu
from jax.experimental.pallas import tpu_sc as plsc

N = 50000
NP = 50176
E = 800000
G = 512
FC = 32
NC, NS = 2, 16
NW = NC * NS
EPT = E // NW
CHUNK = 125
NCHUNK = EPT // CHUNK
RPT = NP // NS
ZROWS = 196
KB = 20
NBUF = 4


def _agg_body(nb, src_hbm, dst_hbm, *rest):
    g_refs = rest[:nb]
    out_ref = rest[nb]
    (sidx, didx, rows0, rows1, rows2, rows3, zbuf, acc,
     gsem, ssem, csem) = rest[nb + 1:]
    rowbufs = (rows0, rows1, rows2, rows3)
    gsems = tuple(gsem.at[b] for b in range(NBUF))
    ssems = tuple(ssem.at[b] for b in range(NBUF))
    c = lax.axis_index("c")
    s = lax.axis_index("s")
    w = c * NS + s

    zeros16 = jnp.zeros((16,), jnp.float32)

    def _zb(i, carry):
        zbuf[i, pl.ds(0, 16)] = zeros16
        zbuf[i, pl.ds(16, 16)] = zeros16
        return carry

    lax.fori_loop(0, ZROWS, _zb, 0)

    for j in range(nb):
        for z in range(RPT // ZROWS):
            pltpu.sync_copy(zbuf, acc.at[pl.ds(s * RPT + z * ZROWS, ZROWS)])
        plsc.subcore_barrier()

        g_ref = g_refs[j]

        def _kblock(kb, carry):
            pltpu.async_copy(src_hbm.at[w, pl.ds(kb * KB, KB)], sidx, csem)
            pltpu.async_copy(dst_hbm.at[w, pl.ds(kb * KB, KB)], didx,
                             csem).wait()
            pltpu.make_async_copy(src_hbm.at[w, pl.ds(kb * KB, KB)], sidx,
                                  csem).wait()

            def _batch(p, c2):
                i0 = p * NBUF
                gathers = []
                for b in range(NBUF):
                    @pl.when(p > 0)
                    def _drain(b=b, i0=i0):
                        pltpu.make_async_copy(
                            rowbufs[b], acc.at[didx.at[i0 + b]],
                            ssems[b]).wait()
                    gathers.append(
                        pltpu.async_copy(g_ref.at[sidx.at[i0 + b]],
                                         rowbufs[b], gsems[b]))
                for b in range(NBUF):
                    gathers[b].wait()
                    pltpu.async_copy(rowbufs[b], acc.at[didx.at[i0 + b]],
                                     ssems[b], add=True)
                return c2

            lax.fori_loop(0, KB // NBUF, _batch, 0)
            last = KB - NBUF
            for b in range(NBUF):
                pltpu.make_async_copy(rowbufs[b], acc.at[didx.at[last + b]],
                                      ssems[b]).wait()
            return carry

        lax.fori_loop(0, NCHUNK // KB, _kblock, 0)
        plsc.subcore_barrier()

        pltpu.sync_copy(acc.at[pl.ds(s * RPT, RPT)],
                        out_ref.at[c, j, pl.ds(s * RPT, RPT)])
        plsc.subcore_barrier()


@functools.lru_cache(maxsize=None)
def _make_agg(nb):
    mesh = plsc.VectorSubcoreMesh(core_axis_name="c", subcore_axis_name="s")
    return pl.kernel(
        functools.partial(_agg_body, nb),
        out_type=jax.ShapeDtypeStruct((NC, nb, NP, FC), jnp.float32),
        mesh=mesh,
        scratch_types=[
            pltpu.VMEM((KB, CHUNK), jnp.int32),
            pltpu.VMEM((KB, CHUNK), jnp.int32),
            pltpu.VMEM((CHUNK, FC), jnp.float32),
            pltpu.VMEM((CHUNK, FC), jnp.float32),
            pltpu.VMEM((CHUNK, FC), jnp.float32),
            pltpu.VMEM((CHUNK, FC), jnp.float32),
            pltpu.VMEM((ZROWS, FC), jnp.float32),
            pltpu.VMEM_SHARED((NP, FC), jnp.float32),
            pltpu.SemaphoreType.DMA((NBUF,)),
            pltpu.SemaphoreType.DMA((NBUF,)),
            pltpu.SemaphoreType.DMA,
        ],
        compiler_params=pltpu.CompilerParams(use_tc_tiling_on_sc=False),
        name="edge_agg_nb%d" % nb,
    )


def _res_tower(x, ws, bs):
    out = jax.nn.relu(jnp.dot(x, ws[0], preferred_element_type=jnp.float32) + bs[0])
    last = out
    for i in range(8):
        if i != 0 and i % 2 == 0:
            out = out + last
        out = jax.nn.relu(jnp.dot(out, ws[1 + i], preferred_element_type=jnp.float32) + bs[1 + i])
        if i != 0 and i % 2 == 0:
            last = out
    return jnp.dot(out, ws[9], preferred_element_type=jnp.float32) + bs[9]


def _heads_kernel(dg_ref, ds_ref, tg_ref, ts_ref, *refs):
    refs = list(refs)
    dsw = [refs[i][...] for i in range(10)]
    dsb = [refs[10 + i][...] for i in range(10)]
    tsw = [refs[20 + i][...] for i in range(10)]
    tsb = [refs[30 + i][...] for i in range(10)]
    dow = [refs[40][...], refs[41][...]]
    dob = [refs[42][...], refs[43][...]]
    tow = [refs[44][...], refs[45][...]]
    tob = [refs[46][...], refs[47][...]]
    d_out_ref, t_out_ref = refs[48], refs[49]

    ds_emb = _res_tower(ds_ref[...], dsw, dsb)
    ts_emb = _res_tower(ts_ref[...], tsw, tsb)

    dcat = jnp.concatenate([dg_ref[...], ds_emb], axis=-1)
    h = jax.nn.relu(jnp.dot(dcat, dow[0], preferred_element_type=jnp.float32) + dob[0])
    d_out_ref[...] = jnp.dot(h, dow[1], preferred_element_type=jnp.float32) + dob[1]

    tcat = jnp.concatenate([tg_ref[...], ts_emb], axis=-1)
    h = jax.nn.relu(jnp.dot(tcat, tow[0], preferred_element_type=jnp.float32) + tob[0])
    t_out_ref[...] = jnp.dot(h, tow[1], preferred_element_type=jnp.float32) + tob[1]


def _run_heads(drug_graph_emb, drug_seq_x, target_graph_emb, target_seq_x, params):
    ds_w = [params['ds_W1']] + [params['ds_Wh%d' % i] for i in range(8)] + [params['ds_W2']]
    ds_b = [params['ds_b1']] + [params['ds_bh%d' % i] for i in range(8)] + [params['ds_b2']]
    ts_w = [params['ts_W1']] + [params['ts_Wh%d' % i] for i in range(8)] + [params['ts_W2']]
    ts_b = [params['ts_b1']] + [params['ts_bh%d' % i] for i in range(8)] + [params['ts_b2']]
    args = ([drug_graph_emb, drug_seq_x, target_graph_emb, target_seq_x]
            + ds_w + ds_b + ts_w + ts_b
            + [params['do_W0'], params['do_W1'], params['do_b0'], params['do_b1']]
            + [params['to_W0'], params['to_W1'], params['to_b0'], params['to_b1']])
    out_shapes = (jax.ShapeDtypeStruct((G, 128), jnp.float32),
                  jax.ShapeDtypeStruct((G, 128), jnp.float32))
    return pl.pallas_call(_heads_kernel, out_shape=out_shapes)(*args)


def _seg_mean_jax(x, seg, n_seg):
    s = jax.ops.segment_sum(x, seg, num_segments=n_seg)
    c = jax.ops.segment_sum(jnp.ones((x.shape[0],), x.dtype), seg,
                            num_segments=n_seg)
    return s / jnp.clip(c, 1.0)[:, None]


RB = 1568


def _split_g(g, nb):
    f = g.shape[1]
    outs = []
    for j in range(nb):
        lo = j * FC
        if lo + FC <= f:
            outs.append(g[:, lo:lo + FC])
        else:
            pad = jnp.zeros((g.shape[0], lo + FC - f), jnp.float32)
            outs.append(jnp.concatenate([g[:, lo:f], pad], axis=-1))
    return outs


def _tc_first_kernel(nb, x_ref, w_ref, dinv_ref, h_ref, *g_refs):
    h = jnp.dot(x_ref[...], w_ref[...], preferred_element_type=jnp.float32)
    h_ref[...] = h
    g = dinv_ref[...] * h
    for j, blk in enumerate(_split_g(g, nb)):
        g_refs[j][...] = blk


def _tc_first(x, w, dinv_col):
    f_out = w.shape[1]
    nb = -(-f_out // FC)
    grid = NP // RB
    out_shape = ([jax.ShapeDtypeStruct((NP, f_out), jnp.float32)]
                 + [jax.ShapeDtypeStruct((NP, FC), jnp.float32)] * nb)
    res = pl.pallas_call(
        functools.partial(_tc_first_kernel, nb),
        grid=(grid,),
        in_specs=[
            pl.BlockSpec((RB, x.shape[1]), lambda r: (r, 0)),
            pl.BlockSpec((w.shape[0], f_out), lambda r: (0, 0)),
            pl.BlockSpec((RB, 1), lambda r: (r, 0)),
        ],
        out_specs=([pl.BlockSpec((RB, f_out), lambda r: (r, 0))]
                   + [pl.BlockSpec((RB, FC), lambda r: (r, 0))] * nb),
        out_shape=out_shape,
    )(x, w, dinv_col)
    return res[0], list(res[1:])


def _tc_combine_kernel(nb_in, nb_out, f_in, p_ref, h_ref, dinv_ref, b_ref,
                       w_ref, hn_ref, *g_refs):
    p = p_ref[...]
    agg = jnp.concatenate([p[0, j] + p[1, j] for j in range(nb_in)],
                          axis=-1)[:, :f_in]
    dinv = dinv_ref[...]
    x = jax.nn.relu(dinv * agg + dinv * dinv * h_ref[...] + b_ref[...])
    hn = jnp.dot(x, w_ref[...], preferred_element_type=jnp.float32)
    hn_ref[...] = hn
    g = dinv * hn
    for j, blk in enumerate(_split_g(g, nb_out)):
        g_refs[j][...] = blk


def _tc_combine(parts, h, dinv_col, b, w):
    nb_in = parts.shape[1]
    f_in = h.shape[1]
    f_out = w.shape[1]
    nb_out = -(-f_out // FC)
    grid = NP // RB
    out_shape = ([jax.ShapeDtypeStruct((NP, f_out), jnp.float32)]
                 + [jax.ShapeDtypeStruct((NP, FC), jnp.float32)] * nb_out)
    res = pl.pallas_call(
        functools.partial(_tc_combine_kernel, nb_in, nb_out, f_in),
        grid=(grid,),
        in_specs=[
            pl.BlockSpec((NC, nb_in, RB, FC), lambda r: (0, 0, r, 0)),
            pl.BlockSpec((RB, f_in), lambda r: (r, 0)),
            pl.BlockSpec((RB, 1), lambda r: (r, 0)),
            pl.BlockSpec((1, f_in), lambda r: (0, 0)),
            pl.BlockSpec((f_in, f_out), lambda r: (0, 0)),
        ],
        out_specs=([pl.BlockSpec((RB, f_out), lambda r: (r, 0))]
                   + [pl.BlockSpec((RB, FC), lambda r: (r, 0))] * nb_out),
        out_shape=out_shape,
    )(parts, h, dinv_col, b, w)
    return res[0], list(res[1:])


def _tc_final_kernel(nb_in, f_in, p_ref, h_ref, dinv_ref, b_ref, x_ref):
    p = p_ref[...]
    agg = jnp.concatenate([p[0, j] + p[1, j] for j in range(nb_in)],
                          axis=-1)[:, :f_in]
    dinv = dinv_ref[...]
    x_ref[...] = jax.nn.relu(dinv * agg + dinv * dinv * h_ref[...]
                             + b_ref[...])


def _tc_final(parts, h, dinv_col, b):
    nb_in = parts.shape[1]
    f_in = h.shape[1]
    grid = NP // RB
    return pl.pallas_call(
        functools.partial(_tc_final_kernel, nb_in, f_in),
        grid=(grid,),
        in_specs=[
            pl.BlockSpec((NC, nb_in, RB, FC), lambda r: (0, 0, r, 0)),
            pl.BlockSpec((RB, f_in), lambda r: (r, 0)),
            pl.BlockSpec((RB, 1), lambda r: (r, 0)),
            pl.BlockSpec((1, f_in), lambda r: (0, 0)),
        ],
        out_specs=pl.BlockSpec((RB, f_in), lambda r: (r, 0)),
        out_shape=jax.ShapeDtypeStruct((NP, f_in), jnp.float32),
    )(parts, h, dinv_col, b)


def _gcn_branch(x, edge_index, params, prefix, dims):
    src = edge_index[0].astype(jnp.int32)
    dst = edge_index[1].astype(jnp.int32)
    deg = jnp.zeros((N,), jnp.float32).at[dst].add(1.0) + 1.0
    dinv = lax.rsqrt(deg)
    dinv_col = jnp.pad(dinv, (0, NP - N), constant_values=1.0)[:, None]
    src2 = src.reshape(NW, NCHUNK, CHUNK)
    dst2 = dst.reshape(NW, NCHUNK, CHUNK)
    x_pad = jnp.pad(x, ((0, NP - N), (0, 0)))
    h, gblocks = _tc_first(x_pad, params['%s_W0' % prefix], dinv_col)
    for i in range(3):
        nb = -(-dims[i + 1] // FC)
        parts = _make_agg(nb)(src2, dst2, *gblocks)
        b = params['%s_b%d' % (prefix, i)][None, :]
        if i < 2:
            h, gblocks = _tc_combine(parts, h, dinv_col, b,
                                     params['%s_W%d' % (prefix, i + 1)])
        else:
            x3 = _tc_final(parts, h, dinv_col, b)
    return x3[:N]


def kernel(drug_x, drug_seq_x, target_x, target_seq_x, params,
           drug_edge_index, drug_batch, target_edge_index, target_batch):
    hd = _gcn_branch(drug_x, drug_edge_index, params, 'dg', [78, 78, 156, 312])
    drug_graph_emb = _seg_mean_jax(hd, drug_batch, G)
    ht = _gcn_branch(target_x, target_edge_index, params, 'tg', [54, 54, 108, 216])
    target_graph_emb = _seg_mean_jax(ht, target_batch, G)
    drug_out, target_out = _run_heads(drug_graph_emb, drug_seq_x,
                                      target_graph_emb, target_seq_x, params)
    return drug_out, target_out
